# baseline (speedup 1.0000x reference)
import functools

import jax
import jax.numpy as jnp
from jax import lax
from jax.experimental import pallas as pl
from jax.experimental.pallas import tpu as pltpu
from jax.experimental.pallas import tpu_sc as plsc

_N = 100000
_D = 64
_L = 128
_NW = 32
_CHUNKS = 24
_NBUF = 4
_BPW = _CHUNKS * _L
_MAIN = _NW * _BPW
_EXTRA = 13
_TAIL = _N - _MAIN - _EXTRA * _L


def _make_gather():
    mesh = plsc.VectorSubcoreMesh(core_axis_name="c", subcore_axis_name="s")

    @functools.partial(
        pl.kernel,
        mesh=mesh,
        out_type=jax.ShapeDtypeStruct((_N, 2 * _D), jnp.float32),
        compiler_params=pltpu.CompilerParams(
            use_tc_tiling_on_sc=True, needs_layout_passes=False),
        scratch_types=(
            [pltpu.VMEM((_BPW,), jnp.int32),
             pltpu.VMEM((_NBUF, _L, 2 * _D), jnp.float32),
             pltpu.VMEM((_L,), jnp.int32),
             pltpu.VMEM((_TAIL,), jnp.int32),
             pltpu.SemaphoreType.DMA]
            + [pltpu.SemaphoreType.DMA] * (2 * _NBUF)
        ),
    )
    def gather_kernel(table2_hbm, pidx_hbm, out_hbm,
                      pidx_v, rows_v, xpidx_v, tpidx_v, xsem, *sems):
        gsem, osem = sems[:_NBUF], sems[_NBUF:]
        wid = lax.axis_index("s") * 2 + lax.axis_index("c")
        base = wid * _BPW
        pltpu.sync_copy(pidx_hbm.at[pl.ds(base, _BPW)], pidx_v)

        gathers = [None] * _NBUF
        outs = [None] * _NBUF
        for g in range(_CHUNKS + _NBUF - 1):
            if g < _CHUNKS:
                b = g % _NBUF
                if g >= _NBUF:
                    outs[b].wait()
                gathers[b] = pltpu.async_copy(
                    table2_hbm.at[pidx_v.at[pl.ds(g * _L, _L)]],
                    rows_v.at[b], gsem[b])
            d = g - (_NBUF - 1)
            if d >= 0:
                db = d % _NBUF
                gathers[db].wait()
                outs[db] = pltpu.async_copy(
                    rows_v.at[db], out_hbm.at[pl.ds(base + d * _L, _L)],
                    osem[db])
        for d in range(max(0, _CHUNKS - _NBUF), _CHUNKS):
            outs[d % _NBUF].wait()

        @pl.when(wid < _EXTRA)
        def _extra():
            start = _MAIN + wid * _L
            pltpu.sync_copy(pidx_hbm.at[pl.ds(start, _L)], xpidx_v)
            pltpu.async_copy(
                table2_hbm.at[xpidx_v], rows_v.at[0], xsem).wait()
            pltpu.sync_copy(rows_v.at[0], out_hbm.at[pl.ds(start, _L)])

        @pl.when(wid == _EXTRA)
        def _tail():
            start = _MAIN + _EXTRA * _L
            pltpu.sync_copy(pidx_hbm.at[pl.ds(start, _TAIL)], tpidx_v)
            pltpu.async_copy(
                table2_hbm.at[tpidx_v], rows_v.at[0, pl.ds(0, _TAIL)],
                xsem).wait()
            pltpu.sync_copy(rows_v.at[0, pl.ds(0, _TAIL)],
                            out_hbm.at[pl.ds(start, _TAIL)])

    return gather_kernel


_gather = _make_gather()


def kernel(node_ids, x, etypes, norm, table):
    del x, etypes, norm
    idx = node_ids.astype(jnp.int32)
    table2 = table.reshape(500000, 2 * _D)
    pairs = _gather(table2, idx >> 1)
    lo = (idx & 1)[:, None]
    out = jnp.where(lo == 0, pairs[:, :_D], pairs[:, _D:])
    return out
